# Initial kernel scaffold; baseline (speedup 1.0000x reference)
import jax
import jax.numpy as jnp
from jax.experimental import pallas as pl


def kernel(x, edge_index, W1, b1, W2, b2):
    raise NotImplementedError("write your pallas kernel here")


# speedup vs baseline: 24.0360x; 24.0360x over previous
import functools

import jax
import jax.numpy as jnp
from jax import lax
from jax.experimental import pallas as pl
from jax.experimental.pallas import tpu as pltpu
from jax.experimental.pallas import tpu_sc as plsc

_NC = 2
_NS = 16
_NW = _NC * _NS
_CH = 125
_NB = 16
_LANES = 16
_ZR = 40


def _mesh():
    return plsc.VectorSubcoreMesh(core_axis_name="c", subcore_axis_name="s")


def _make_deg_kernel(n_nodes, n_edges):
    nchunk = (n_edges // _NW) // _CH
    nz = n_nodes // _ZR
    nz_per_tile = -(-nz // _NS)

    @functools.partial(
        pl.kernel,
        out_type=jax.ShapeDtypeStruct((_NC, n_nodes, _LANES), jnp.float32),
        mesh=_mesh(),
        scratch_types=[
            pltpu.VMEM((nchunk, _CH), jnp.int32),
            pltpu.VMEM((_CH, _LANES), jnp.float32),
            pltpu.VMEM((_ZR, _LANES), jnp.float32),
            pltpu.VMEM_SHARED((n_nodes, _LANES), jnp.float32),
        ],
    )
    def deg_kernel(dst_hbm, out_hbm, dst_v, ones_v, zbuf, acc_sh):
        c = lax.axis_index("c")
        s = lax.axis_index("s")
        wid = c * _NS + s
        pltpu.sync_copy(dst_hbm.at[pl.ds(wid * nchunk, nchunk)], dst_v)

        z16 = jnp.zeros((_LANES,), jnp.float32)
        o16 = jnp.ones((_LANES,), jnp.float32)

        def fill_ones(i, carry):
            ones_v[i, pl.ds(0, _LANES)] = o16
            return carry

        lax.fori_loop(0, _CH, fill_ones, None)

        def fill_z(i, carry):
            zbuf[i, pl.ds(0, _LANES)] = z16
            return carry

        lax.fori_loop(0, _ZR, fill_z, None)

        for t in range(nz_per_tile):
            k = s + t * _NS

            @pl.when(k < nz)
            def _():
                pltpu.sync_copy(zbuf, acc_sh.at[pl.ds(k * _ZR, _ZR)])

        plsc.subcore_barrier()

        def body(j, carry):
            pltpu.sync_copy(ones_v, acc_sh.at[dst_v.at[j]], add=True)
            return carry

        lax.fori_loop(0, nchunk, body, None)
        plsc.subcore_barrier()

        for t in range(nz_per_tile):
            k = s + t * _NS

            @pl.when(k < nz)
            def _():
                pltpu.sync_copy(acc_sh.at[pl.ds(k * _ZR, _ZR)], zbuf)
                pltpu.sync_copy(zbuf, out_hbm.at[c, pl.ds(k * _ZR, _ZR)])

    return deg_kernel


def _make_agg_kernel(n_nodes, n_edges, d):
    nchunk = (n_edges // _NW) // _CH
    nz = n_nodes // _ZR
    nz_per_tile = -(-nz // _NS)

    nblk = nchunk // _NB

    @functools.partial(
        pl.kernel,
        out_type=jax.ShapeDtypeStruct((_NC, n_nodes, d), jnp.float32),
        mesh=_mesh(),
        scratch_types=[
            pltpu.VMEM((_NB, _CH), jnp.int32),
            pltpu.VMEM((_NB, _CH), jnp.int32),
            pltpu.VMEM((_CH, d), jnp.float32),
            pltpu.VMEM((_CH, d), jnp.float32),
            pltpu.VMEM((_ZR, d), jnp.float32),
            pltpu.VMEM_SHARED((n_nodes, d), jnp.float32),
            pltpu.SemaphoreType.DMA,
            pltpu.SemaphoreType.DMA,
        ],
    )
    def agg(xs_hbm, src_hbm, dst_hbm, out_hbm,
            src_v, dst_v, buf0, buf1, zbuf, acc_sh, sem0, sem1):
        c = lax.axis_index("c")
        s = lax.axis_index("s")
        wid = c * _NS + s

        z16 = jnp.zeros((_LANES,), jnp.float32)

        def zrow(i, carry):
            def zlane(g, carry2):
                zbuf[i, pl.ds(g * _LANES, _LANES)] = z16
                return carry2

            lax.fori_loop(0, d // _LANES, zlane, None)
            return carry

        lax.fori_loop(0, _ZR, zrow, None)

        for t in range(nz_per_tile):
            k = s + t * _NS

            @pl.when(k < nz)
            def _():
                pltpu.sync_copy(zbuf, acc_sh.at[pl.ds(k * _ZR, _ZR)])

        plsc.subcore_barrier()

        def blk_body(b, carry):
            base = wid * nchunk + b * _NB
            pltpu.sync_copy(src_hbm.at[pl.ds(base, _NB)], src_v)
            pltpu.sync_copy(dst_hbm.at[pl.ds(base, _NB)], dst_v)

            def body(t, carry2):
                j0 = 2 * t
                c0 = pltpu.async_copy(xs_hbm.at[src_v.at[j0]], buf0, sem0)
                c1 = pltpu.async_copy(xs_hbm.at[src_v.at[j0 + 1]], buf1, sem1)
                c0.wait()
                pltpu.sync_copy(buf0, acc_sh.at[dst_v.at[j0]], add=True)
                c1.wait()
                pltpu.sync_copy(buf1, acc_sh.at[dst_v.at[j0 + 1]], add=True)
                return carry2

            lax.fori_loop(0, _NB // 2, body, None)
            return carry

        lax.fori_loop(0, nblk, blk_body, None)

        plsc.subcore_barrier()

        for t in range(nz_per_tile):
            k = s + t * _NS

            @pl.when(k < nz)
            def _():
                pltpu.sync_copy(acc_sh.at[pl.ds(k * _ZR, _ZR)], zbuf)
                pltpu.sync_copy(zbuf, out_hbm.at[c, pl.ds(k * _ZR, _ZR)])

    return agg


_BM = 1000


def _tc_mm_scale(x, w, degp):
    n, din = x.shape
    dout = w.shape[1]

    def body(x_ref, w_ref, dp_ref, xw_ref, xs_ref, d_ref):
        xw = jnp.dot(x_ref[...], w_ref[...], preferred_element_type=jnp.float32)
        dval = lax.rsqrt(dp_ref[0, :, 0:1] + dp_ref[1, :, 0:1] + 1.0)
        xw_ref[...] = xw
        xs_ref[...] = dval * xw
        d_ref[...] = dval

    return pl.pallas_call(
        body,
        grid=(n // _BM,),
        in_specs=[
            pl.BlockSpec((_BM, din), lambda i: (i, 0)),
            pl.BlockSpec((din, dout), lambda i: (0, 0)),
            pl.BlockSpec((_NC, _BM, _LANES), lambda i: (0, i, 0)),
        ],
        out_specs=[
            pl.BlockSpec((_BM, dout), lambda i: (i, 0)),
            pl.BlockSpec((_BM, dout), lambda i: (i, 0)),
            pl.BlockSpec((_BM, 1), lambda i: (i, 0)),
        ],
        out_shape=[
            jax.ShapeDtypeStruct((n, dout), jnp.float32),
            jax.ShapeDtypeStruct((n, dout), jnp.float32),
            jax.ShapeDtypeStruct((n, 1), jnp.float32),
        ],
    )(x, w, degp)


def _tc_combine_relu(p, xw, dvec, b1):
    n, dh = xw.shape

    def body(p_ref, xw_ref, d_ref, b1_ref, h_ref, xs2_ref):
        dval = d_ref[...]
        agg = p_ref[0] + p_ref[1]
        h = jnp.maximum(dval * agg + (dval * dval) * xw_ref[...] + b1_ref[...], 0.0)
        h_ref[...] = h
        xs2_ref[...] = dval * h

    return pl.pallas_call(
        body,
        grid=(n // _BM,),
        in_specs=[
            pl.BlockSpec((_NC, _BM, dh), lambda i: (0, i, 0)),
            pl.BlockSpec((_BM, dh), lambda i: (i, 0)),
            pl.BlockSpec((_BM, 1), lambda i: (i, 0)),
            pl.BlockSpec((1, dh), lambda i: (0, 0)),
        ],
        out_specs=[
            pl.BlockSpec((_BM, dh), lambda i: (i, 0)),
            pl.BlockSpec((_BM, dh), lambda i: (i, 0)),
        ],
        out_shape=[
            jax.ShapeDtypeStruct((n, dh), jnp.float32),
            jax.ShapeDtypeStruct((n, dh), jnp.float32),
        ],
    )(p, xw, dvec, b1)


def _tc_final_mm(q, h, dvec, w2, b2):
    n, dh = h.shape
    dout = w2.shape[1]

    def body(q_ref, h_ref, d_ref, w2_ref, b2_ref, o_ref):
        dval = d_ref[...]
        agg2 = dval * (q_ref[0] + q_ref[1]) + (dval * dval) * h_ref[...]
        o_ref[...] = (jnp.dot(agg2, w2_ref[...], preferred_element_type=jnp.float32)
                      + b2_ref[...])

    return pl.pallas_call(
        body,
        grid=(n // _BM,),
        in_specs=[
            pl.BlockSpec((_NC, _BM, dh), lambda i: (0, i, 0)),
            pl.BlockSpec((_BM, dh), lambda i: (i, 0)),
            pl.BlockSpec((_BM, 1), lambda i: (i, 0)),
            pl.BlockSpec((dh, dout), lambda i: (0, 0)),
            pl.BlockSpec((1, dout), lambda i: (0, 0)),
        ],
        out_specs=pl.BlockSpec((_BM, dout), lambda i: (i, 0)),
        out_shape=jax.ShapeDtypeStruct((n, dout), jnp.float32),
    )(q, h, dvec, w2, b2)


def kernel(x, edge_index, W1, b1, W2, b2):
    n, _ = x.shape
    e = edge_index.shape[1]
    dhid = W1.shape[1]
    ncls = W2.shape[1]

    src = edge_index[0].astype(jnp.int32)
    dst = edge_index[1].astype(jnp.int32)
    src2 = src.reshape(e // _CH, _CH)
    dst2 = dst.reshape(e // _CH, _CH)

    degp = _make_deg_kernel(n, e)(dst2)

    xw, xs1, dvec = _tc_mm_scale(x, W1, degp)
    agg = _make_agg_kernel(n, e, dhid)
    p = agg(xs1, src2, dst2)
    h, xs2 = _tc_combine_relu(p, xw, dvec, b1.reshape(1, dhid))
    q = agg(xs2, src2, dst2)
    out = _tc_final_mm(q, h, dvec, W2, b2.reshape(1, ncls))
    out = lax.optimization_barrier((out, degp, xw, xs1, dvec, p, h, xs2, q))[0]
    return out
